# baseline (speedup 1.0000x reference)
import jax
import jax.numpy as jnp
from jax import lax
from jax.experimental import pallas as pl
from jax.experimental.pallas import tpu as pltpu
from jax.experimental.pallas import tpu_sc as plsc

N = 10000
E = 320000
D = 128
DE = 16

NC = 2
NS = 16
NW = NC * NS
CK = 5
ECH = E // CK
EPWC = ECH // NW
SBATCH = 40
SNBC = EPWC // SBATCH
NBUF = 5
NP = 10240
RPS = NP // NS

_sc_mesh = plsc.VectorSubcoreMesh(core_axis_name="c", subcore_axis_name="s")


def _gather_body(table_hbm, idx_hbm, out_hbm, ibuf, rows, table_s, wsems, isems):
    c = lax.axis_index("c")
    s = lax.axis_index("s")
    wid = s * NC + c
    base = wid * EPWC

    def icopy(j, b):
        return pltpu.make_async_copy(idx_hbm.at[wid, j], ibuf.at[b], isems.at[b])

    for b in range(NBUF):
        icopy(b, b).start()

    @pl.when(s < NS - 1)
    def _():
        pltpu.sync_copy(table_hbm.at[pl.ds(s * 640, 640)],
                        table_s.at[pl.ds(s * 640, 640)])

    @pl.when(s == NS - 1)
    def _():
        pltpu.sync_copy(table_hbm.at[pl.ds(9600, N - 9600)],
                        table_s.at[pl.ds(9600, N - 9600)])

    plsc.subcore_barrier()

    def wcopy(j, b):
        return pltpu.make_async_copy(
            rows.at[b], out_hbm.at[pl.ds(base + j * SBATCH, SBATCH)], wsems.at[b]
        )

    @pl.loop(0, SNBC, step=NBUF)
    def _outer(i):
        for k in range(NBUF):
            j = i + k

            @pl.when(j >= NBUF)
            def _():
                wcopy(j - NBUF, k).wait()

            icopy(j, k).wait()
            pltpu.sync_copy(table_s.at[ibuf.at[k]], rows.at[k])
            nj = j + NBUF

            @pl.when(nj < SNBC)
            def _():
                icopy(nj, k).start()

            wcopy(j, k).start()

    for k in range(NBUF):
        wcopy(SNBC - NBUF + k, k).wait()


_gather = pl.kernel(
    _gather_body,
    out_type=jax.ShapeDtypeStruct((ECH, D), jnp.float32),
    mesh=_sc_mesh,
    scratch_types=[
        pltpu.VMEM((NBUF, SBATCH), jnp.int32),
        pltpu.VMEM((NBUF, SBATCH, D), jnp.float32),
        pltpu.VMEM_SHARED((N, D), jnp.float32),
        pltpu.SemaphoreType.DMA((NBUF,)),
        pltpu.SemaphoreType.DMA((NBUF,)),
    ],
)


def _make_scatter(nchunks):
    def body(*refs):
        ms = refs[:nchunks]
        dst_hbm = refs[nchunks]
        zeros_hbm = refs[nchunks + 1]
        out_hbm = refs[nchunks + 2]
        ibuf, rows, acc, msems, isems = refs[nchunks + 3:]
        c = lax.axis_index("c")
        s = lax.axis_index("s")
        wid = s * NC + c
        base = wid * EPWC
        pltpu.sync_copy(zeros_hbm, acc.at[pl.ds(s * RPS, RPS)])
        plsc.subcore_barrier()

        for kc in range(nchunks):
            def mcopy(j, b, kc=kc):
                return pltpu.make_async_copy(
                    ms[kc].at[pl.ds(base + j * SBATCH, SBATCH)],
                    rows.at[b], msems.at[b]
                )

            def icopy(j, b, kc=kc):
                return pltpu.make_async_copy(
                    dst_hbm.at[kc, wid, j], ibuf.at[b], isems.at[b]
                )

            for b in range(NBUF):
                mcopy(b, b).start()
                icopy(b, b).start()

            @pl.loop(0, SNBC, step=NBUF)
            def _outer(i):
                for k in range(NBUF):
                    j = i + k
                    mcopy(j, k).wait()
                    icopy(j, k).wait()
                    pltpu.sync_copy(rows.at[k], acc.at[ibuf.at[k]], add=True)
                    nj = j + NBUF

                    @pl.when(nj < SNBC)
                    def _():
                        mcopy(nj, k).start()
                        icopy(nj, k).start()

        plsc.subcore_barrier()
        pltpu.sync_copy(acc.at[pl.ds(s * RPS, RPS)],
                        out_hbm.at[c, pl.ds(s * RPS, RPS)])

    return pl.kernel(
        body,
        out_type=jax.ShapeDtypeStruct((NC, NP, D), jnp.float32),
        mesh=_sc_mesh,
        scratch_types=[
            pltpu.VMEM((NBUF, SBATCH), jnp.int32),
            pltpu.VMEM((NBUF, SBATCH, D), jnp.float32),
            pltpu.VMEM_SHARED((NP, D), jnp.float32),
            pltpu.SemaphoreType.DMA((NBUF,)),
            pltpu.SemaphoreType.DMA((NBUF,)),
        ],
    )


_scatter_a = _make_scatter(3)
_scatter_b = _make_scatter(2)


def _a1_body(h_ref, w_ref, b_ref, o_ref):
    o_ref[...] = (
        jnp.dot(h_ref[...], w_ref[...], preferred_element_type=jnp.float32)
        + b_ref[...]
    )


def _mlp_body(aj_ref, ea_ref, w1b_ref, w2_ref, b2_ref, o_ref):
    x = aj_ref[...] + jnp.dot(
        ea_ref[...], w1b_ref[...], preferred_element_type=jnp.float32
    )
    h = 0.5 * x * (1.0 + lax.erf(x * 0.7071067811865476))
    o_ref[...] = (
        jnp.dot(h.astype(jnp.bfloat16), w2_ref[...],
                preferred_element_type=jnp.float32)
        + b2_ref[...]
    )


def _ln_body(h_ref, agg_a_ref, agg_b_ref, g_ref, beta_ref, o_ref):
    x = (h_ref[...] + agg_a_ref[0] + agg_a_ref[1]
         + agg_b_ref[0] + agg_b_ref[1])
    mu = jnp.mean(x, axis=-1, keepdims=True)
    xc = x - mu
    var = jnp.mean(xc * xc, axis=-1, keepdims=True)
    o_ref[...] = xc * lax.rsqrt(var + 1e-5) * g_ref[...] + beta_ref[...]


_NBLK = 1000
_EBLK = 4000


def kernel(H, edge_index, edge_attr, W1, b1, W2, b2, gamma, beta):
    src = edge_index[0].astype(jnp.int32).reshape(CK, NW, SNBC, SBATCH)
    dst = edge_index[1].astype(jnp.int32).reshape(CK, NW, SNBC, SBATCH)
    W1a = W1[:D]
    W1b = W1[D:]
    W2b = W2.astype(jnp.bfloat16)
    b1r = b1.reshape(1, D)
    b2r = b2.reshape(1, D)
    gr = gamma.reshape(1, D)
    br = beta.reshape(1, D)
    zrows = jnp.zeros((RPS, D), jnp.float32)

    A1 = pl.pallas_call(
        _a1_body,
        grid=(N // _NBLK,),
        in_specs=[
            pl.BlockSpec((_NBLK, D), lambda i: (i, 0)),
            pl.BlockSpec((D, D), lambda i: (0, 0)),
            pl.BlockSpec((1, D), lambda i: (0, 0)),
        ],
        out_specs=pl.BlockSpec((_NBLK, D), lambda i: (i, 0)),
        out_shape=jax.ShapeDtypeStruct((N, D), jnp.float32),
    )(H, W1a, b1r)

    ms = []
    for kc in range(CK):
        Aj = _gather(A1, src[kc])
        off = kc * (ECH // _EBLK)
        ms.append(pl.pallas_call(
            _mlp_body,
            grid=(ECH // _EBLK,),
            in_specs=[
                pl.BlockSpec((_EBLK, D), lambda i: (i, 0)),
                pl.BlockSpec((_EBLK, DE), lambda i, off=off: (off + i, 0)),
                pl.BlockSpec((DE, D), lambda i: (0, 0)),
                pl.BlockSpec((D, D), lambda i: (0, 0)),
                pl.BlockSpec((1, D), lambda i: (0, 0)),
            ],
            out_specs=pl.BlockSpec((_EBLK, D), lambda i: (i, 0)),
            out_shape=jax.ShapeDtypeStruct((ECH, D), jnp.float32),
        )(Aj, edge_attr, W1b, W2b, b2r))

    agg_a = _scatter_a(ms[0], ms[1], ms[2], dst[:3], zrows)
    agg_b = _scatter_b(ms[3], ms[4], dst[3:], zrows)

    out = pl.pallas_call(
        _ln_body,
        grid=(N // _NBLK,),
        in_specs=[
            pl.BlockSpec((_NBLK, D), lambda i: (i, 0)),
            pl.BlockSpec((NC, _NBLK, D), lambda i: (0, i, 0)),
            pl.BlockSpec((NC, _NBLK, D), lambda i: (0, i, 0)),
            pl.BlockSpec((1, D), lambda i: (0, 0)),
            pl.BlockSpec((1, D), lambda i: (0, 0)),
        ],
        out_specs=pl.BlockSpec((_NBLK, D), lambda i: (i, 0)),
        out_shape=jax.ShapeDtypeStruct((N, D), jnp.float32),
    )(H, agg_a, agg_b, gr, br)

    return out


# speedup vs baseline: 1.2240x; 1.2240x over previous
import jax
import jax.numpy as jnp
from jax import lax
from jax.experimental import pallas as pl
from jax.experimental.pallas import tpu as pltpu
from jax.experimental.pallas import tpu_sc as plsc

N = 10000
E = 320000
D = 128
DE = 16

NC = 2
NS = 16
NW = NC * NS
CK = 5
ECH = E // CK
EPWC = ECH // NW
SBATCH = 40
SNBC = EPWC // SBATCH
NBUF = 5
NP = 10240
RPS = NP // NS

_sc_mesh = plsc.VectorSubcoreMesh(core_axis_name="c", subcore_axis_name="s")


def _gather_body(table_hbm, idx_hbm, out_hbm, ibuf, rows, table_s, wsems, isems):
    c = lax.axis_index("c")
    s = lax.axis_index("s")
    wid = s * NC + c
    base = wid * EPWC

    def icopy(j, b):
        return pltpu.make_async_copy(idx_hbm.at[wid, j], ibuf.at[b], isems.at[b])

    for b in range(NBUF):
        icopy(b, b).start()

    @pl.when(s < NS - 1)
    def _():
        pltpu.sync_copy(table_hbm.at[pl.ds(s * 640, 640)],
                        table_s.at[pl.ds(s * 640, 640)])

    @pl.when(s == NS - 1)
    def _():
        pltpu.sync_copy(table_hbm.at[pl.ds(9600, N - 9600)],
                        table_s.at[pl.ds(9600, N - 9600)])

    plsc.subcore_barrier()

    def wcopy(j, b):
        return pltpu.make_async_copy(
            rows.at[b], out_hbm.at[pl.ds(base + j * SBATCH, SBATCH)], wsems.at[b]
        )

    @pl.loop(0, SNBC, step=NBUF)
    def _outer(i):
        for k in range(NBUF):
            j = i + k

            @pl.when(j >= NBUF)
            def _():
                wcopy(j - NBUF, k).wait()

            icopy(j, k).wait()
            pltpu.sync_copy(table_s.at[ibuf.at[k]], rows.at[k])
            nj = j + NBUF

            @pl.when(nj < SNBC)
            def _():
                icopy(nj, k).start()

            wcopy(j, k).start()

    for k in range(NBUF):
        wcopy(SNBC - NBUF + k, k).wait()


_gather = pl.kernel(
    _gather_body,
    out_type=jax.ShapeDtypeStruct((ECH, D), jnp.float32),
    mesh=_sc_mesh,
    scratch_types=[
        pltpu.VMEM((NBUF, SBATCH), jnp.int32),
        pltpu.VMEM((NBUF, SBATCH, D), jnp.float32),
        pltpu.VMEM_SHARED((N, D), jnp.float32),
        pltpu.SemaphoreType.DMA((NBUF,)),
        pltpu.SemaphoreType.DMA((NBUF,)),
    ],
)


def _make_scatter(nchunks):
    def body(*refs):
        ms = refs[:nchunks]
        dst_hbm = refs[nchunks]
        zeros_hbm = refs[nchunks + 1]
        out_hbm = refs[nchunks + 2]
        ibuf, rows, acc, msems, isems = refs[nchunks + 3:]
        c = lax.axis_index("c")
        s = lax.axis_index("s")
        wid = s * NC + c
        base = wid * EPWC
        pltpu.sync_copy(zeros_hbm, acc.at[pl.ds(s * RPS, RPS)])
        plsc.subcore_barrier()

        for kc in range(nchunks):
            def mcopy(j, b, kc=kc):
                return pltpu.make_async_copy(
                    ms[kc].at[pl.ds(base + j * SBATCH, SBATCH)],
                    rows.at[b], msems.at[b]
                )

            def icopy(j, b, kc=kc):
                return pltpu.make_async_copy(
                    dst_hbm.at[kc, wid, j], ibuf.at[b], isems.at[b]
                )

            for b in range(NBUF):
                mcopy(b, b).start()
                icopy(b, b).start()

            @pl.loop(0, SNBC, step=NBUF)
            def _outer(i):
                for k in range(NBUF):
                    j = i + k
                    mcopy(j, k).wait()
                    icopy(j, k).wait()
                    pltpu.sync_copy(rows.at[k], acc.at[ibuf.at[k]], add=True)
                    nj = j + NBUF

                    @pl.when(nj < SNBC)
                    def _():
                        mcopy(nj, k).start()
                        icopy(nj, k).start()

        plsc.subcore_barrier()
        pltpu.sync_copy(acc.at[pl.ds(s * RPS, RPS)],
                        out_hbm.at[c, pl.ds(s * RPS, RPS)])

    return pl.kernel(
        body,
        out_type=jax.ShapeDtypeStruct((NC, NP, D), jnp.float32),
        mesh=_sc_mesh,
        scratch_types=[
            pltpu.VMEM((NBUF, SBATCH), jnp.int32),
            pltpu.VMEM((NBUF, SBATCH, D), jnp.float32),
            pltpu.VMEM_SHARED((NP, D), jnp.float32),
            pltpu.SemaphoreType.DMA((NBUF,)),
            pltpu.SemaphoreType.DMA((NBUF,)),
        ],
    )


_scatter_ab = _make_scatter(2)
_scatter_c = _make_scatter(1)


def _a1_body(h_ref, w_ref, b_ref, o_ref):
    o_ref[...] = (
        jnp.dot(h_ref[...], w_ref[...], preferred_element_type=jnp.float32)
        + b_ref[...]
    )


def _mlp_body(aj_ref, ea_ref, w1b_ref, w2_ref, b2_ref, o_ref):
    x = aj_ref[...] + lax.dot_general(
        ea_ref[...], w1b_ref[...], (((0,), (0,)), ((), ())),
        preferred_element_type=jnp.float32,
    )
    h = 0.5 * x * (1.0 + lax.erf(x * 0.7071067811865476))
    o_ref[...] = (
        jnp.dot(h.astype(jnp.bfloat16), w2_ref[...],
                preferred_element_type=jnp.float32)
        + b2_ref[...]
    )


def _ln_body(h_ref, agg_a_ref, agg_b_ref, agg_c_ref, g_ref, beta_ref, o_ref):
    x = (h_ref[...] + agg_a_ref[0] + agg_a_ref[1]
         + agg_b_ref[0] + agg_b_ref[1] + agg_c_ref[0] + agg_c_ref[1])
    mu = jnp.mean(x, axis=-1, keepdims=True)
    xc = x - mu
    var = jnp.mean(xc * xc, axis=-1, keepdims=True)
    o_ref[...] = xc * lax.rsqrt(var + 1e-5) * g_ref[...] + beta_ref[...]


_NBLK = 1000
_EBLK = 6400


def kernel(H, edge_index, edge_attr, W1, b1, W2, b2, gamma, beta):
    src = edge_index[0].astype(jnp.int32).reshape(CK, NW, SNBC, SBATCH)
    dst = edge_index[1].astype(jnp.int32).reshape(CK, NW, SNBC, SBATCH)
    W1a = W1[:D]
    W1b = W1[D:]
    W2b = W2.astype(jnp.bfloat16)
    b1r = b1.reshape(1, D)
    b2r = b2.reshape(1, D)
    gr = gamma.reshape(1, D)
    br = beta.reshape(1, D)
    zrows = jnp.zeros((RPS, D), jnp.float32)
    ea_t = edge_attr.T

    A1 = pl.pallas_call(
        _a1_body,
        grid=(N // _NBLK,),
        in_specs=[
            pl.BlockSpec((_NBLK, D), lambda i: (i, 0)),
            pl.BlockSpec((D, D), lambda i: (0, 0)),
            pl.BlockSpec((1, D), lambda i: (0, 0)),
        ],
        out_specs=pl.BlockSpec((_NBLK, D), lambda i: (i, 0)),
        out_shape=jax.ShapeDtypeStruct((N, D), jnp.float32),
    )(H, W1a, b1r)

    ms = []
    for kc in range(CK):
        Aj = _gather(A1, src[kc])
        off = kc * (ECH // _EBLK)
        ms.append(pl.pallas_call(
            _mlp_body,
            grid=(ECH // _EBLK,),
            in_specs=[
                pl.BlockSpec((_EBLK, D), lambda i: (i, 0)),
                pl.BlockSpec((DE, _EBLK), lambda i, off=off: (0, off + i)),
                pl.BlockSpec((DE, D), lambda i: (0, 0)),
                pl.BlockSpec((D, D), lambda i: (0, 0)),
                pl.BlockSpec((1, D), lambda i: (0, 0)),
            ],
            out_specs=pl.BlockSpec((_EBLK, D), lambda i: (i, 0)),
            out_shape=jax.ShapeDtypeStruct((ECH, D), jnp.float32),
        )(Aj, ea_t, W1b, W2b, b2r))

    agg_a = _scatter_ab(ms[0], ms[1], dst[0:2], zrows)
    agg_b = _scatter_ab(ms[2], ms[3], dst[2:4], zrows)
    agg_c = _scatter_c(ms[4], dst[4:5], zrows)

    out = pl.pallas_call(
        _ln_body,
        grid=(N // _NBLK,),
        in_specs=[
            pl.BlockSpec((_NBLK, D), lambda i: (i, 0)),
            pl.BlockSpec((NC, _NBLK, D), lambda i: (0, i, 0)),
            pl.BlockSpec((NC, _NBLK, D), lambda i: (0, i, 0)),
            pl.BlockSpec((NC, _NBLK, D), lambda i: (0, i, 0)),
            pl.BlockSpec((1, D), lambda i: (0, 0)),
            pl.BlockSpec((1, D), lambda i: (0, 0)),
        ],
        out_specs=pl.BlockSpec((_NBLK, D), lambda i: (i, 0)),
        out_shape=jax.ShapeDtypeStruct((N, D), jnp.float32),
    )(H, agg_a, agg_b, agg_c, gr, br)

    return out


# speedup vs baseline: 1.3218x; 1.0799x over previous
import jax
import jax.numpy as jnp
from jax import lax
from jax.experimental import pallas as pl
from jax.experimental.pallas import tpu as pltpu
from jax.experimental.pallas import tpu_sc as plsc

N = 10000
E = 320000
D = 128
DE = 16

NC = 2
NS = 16
NW = NC * NS
CK = 5
ECH = E // CK
EPWC = ECH // NW
SBATCH = 40
SNBC = EPWC // SBATCH
NBUF = 5
NP = 10240
RPS = NP // NS

_sc_mesh = plsc.VectorSubcoreMesh(core_axis_name="c", subcore_axis_name="s")


def _make_gather(nchunks):
    def body(table_hbm, idx_hbm, out_hbm, ibuf, rows, table_s, wsems, isems):
        c = lax.axis_index("c")
        s = lax.axis_index("s")
        wid = s * NC + c
        base = wid * EPWC

        @pl.when(s < NS - 1)
        def _():
            pltpu.sync_copy(table_hbm.at[pl.ds(s * 640, 640)],
                            table_s.at[pl.ds(s * 640, 640)])

        @pl.when(s == NS - 1)
        def _():
            pltpu.sync_copy(table_hbm.at[pl.ds(9600, N - 9600)],
                            table_s.at[pl.ds(9600, N - 9600)])

        plsc.subcore_barrier()

        for kc in range(nchunks):
            def icopy(j, b, kc=kc):
                return pltpu.make_async_copy(
                    idx_hbm.at[kc, wid, j], ibuf.at[b], isems.at[b]
                )

            def wcopy(j, b, kc=kc):
                return pltpu.make_async_copy(
                    rows.at[b],
                    out_hbm.at[pl.ds(kc * ECH + base + j * SBATCH, SBATCH)],
                    wsems.at[b],
                )

            for b in range(NBUF):
                icopy(b, b).start()

            @pl.loop(0, SNBC, step=NBUF)
            def _outer(i):
                for k in range(NBUF):
                    j = i + k

                    @pl.when(j >= NBUF)
                    def _():
                        wcopy(j - NBUF, k).wait()

                    icopy(j, k).wait()
                    pltpu.sync_copy(table_s.at[ibuf.at[k]], rows.at[k])
                    nj = j + NBUF

                    @pl.when(nj < SNBC)
                    def _():
                        icopy(nj, k).start()

                    wcopy(j, k).start()

            for k in range(NBUF):
                wcopy(SNBC - NBUF + k, k).wait()

    return pl.kernel(
        body,
        out_type=jax.ShapeDtypeStruct((nchunks * ECH, D), jnp.float32),
        mesh=_sc_mesh,
        scratch_types=[
            pltpu.VMEM((NBUF, SBATCH), jnp.int32),
            pltpu.VMEM((NBUF, SBATCH, D), jnp.float32),
            pltpu.VMEM_SHARED((N, D), jnp.float32),
            pltpu.SemaphoreType.DMA((NBUF,)),
            pltpu.SemaphoreType.DMA((NBUF,)),
        ],
    )


_gather_a = _make_gather(2)
_gather_b = _make_gather(3)


def _make_scatter(nchunks):
    def body(*refs):
        ms = refs[:nchunks]
        dst_hbm = refs[nchunks]
        zeros_hbm = refs[nchunks + 1]
        out_hbm = refs[nchunks + 2]
        ibuf, rows, acc, msems, isems = refs[nchunks + 3:]
        c = lax.axis_index("c")
        s = lax.axis_index("s")
        wid = s * NC + c
        base = wid * EPWC
        pltpu.sync_copy(zeros_hbm, acc.at[pl.ds(s * RPS, RPS)])
        plsc.subcore_barrier()

        for kc in range(nchunks):
            def mcopy(j, b, kc=kc):
                return pltpu.make_async_copy(
                    ms[kc].at[pl.ds(base + j * SBATCH, SBATCH)],
                    rows.at[b], msems.at[b]
                )

            def icopy(j, b, kc=kc):
                return pltpu.make_async_copy(
                    dst_hbm.at[kc, wid, j], ibuf.at[b], isems.at[b]
                )

            for b in range(NBUF):
                mcopy(b, b).start()
                icopy(b, b).start()

            @pl.loop(0, SNBC, step=NBUF)
            def _outer(i):
                for k in range(NBUF):
                    j = i + k
                    mcopy(j, k).wait()
                    icopy(j, k).wait()
                    pltpu.sync_copy(rows.at[k], acc.at[ibuf.at[k]], add=True)
                    nj = j + NBUF

                    @pl.when(nj < SNBC)
                    def _():
                        mcopy(nj, k).start()
                        icopy(nj, k).start()

        plsc.subcore_barrier()
        pltpu.sync_copy(acc.at[pl.ds(s * RPS, RPS)],
                        out_hbm.at[c, pl.ds(s * RPS, RPS)])

    return pl.kernel(
        body,
        out_type=jax.ShapeDtypeStruct((NC, NP, D), jnp.float32),
        mesh=_sc_mesh,
        scratch_types=[
            pltpu.VMEM((NBUF, SBATCH), jnp.int32),
            pltpu.VMEM((NBUF, SBATCH, D), jnp.float32),
            pltpu.VMEM_SHARED((NP, D), jnp.float32),
            pltpu.SemaphoreType.DMA((NBUF,)),
            pltpu.SemaphoreType.DMA((NBUF,)),
        ],
    )


_scatter_ab = _make_scatter(2)
_scatter_c = _make_scatter(1)


def _a1_body(h_ref, w_ref, b_ref, o_ref):
    o_ref[...] = (
        jnp.dot(h_ref[...], w_ref[...], preferred_element_type=jnp.float32)
        + b_ref[...]
    )


def _mlp_body(aj_ref, ea_ref, w1b_ref, w2_ref, b2_ref, o_ref):
    x = aj_ref[...] + lax.dot_general(
        ea_ref[...], w1b_ref[...], (((0,), (0,)), ((), ())),
        preferred_element_type=jnp.float32,
    )
    h = 0.5 * x * (1.0 + lax.erf(x * 0.7071067811865476))
    o_ref[...] = (
        jnp.dot(h.astype(jnp.bfloat16), w2_ref[...],
                preferred_element_type=jnp.float32)
        + b2_ref[...]
    )


def _ln_body(h_ref, agg_a_ref, agg_b_ref, agg_c_ref, g_ref, beta_ref, o_ref):
    x = (h_ref[...] + agg_a_ref[0] + agg_a_ref[1]
         + agg_b_ref[0] + agg_b_ref[1] + agg_c_ref[0] + agg_c_ref[1])
    mu = jnp.mean(x, axis=-1, keepdims=True)
    xc = x - mu
    var = jnp.mean(xc * xc, axis=-1, keepdims=True)
    o_ref[...] = xc * lax.rsqrt(var + 1e-5) * g_ref[...] + beta_ref[...]


_NBLK = 1000
_EBLK = 6400


def kernel(H, edge_index, edge_attr, W1, b1, W2, b2, gamma, beta):
    src = edge_index[0].astype(jnp.int32).reshape(CK, NW, SNBC, SBATCH)
    dst = edge_index[1].astype(jnp.int32).reshape(CK, NW, SNBC, SBATCH)
    W1a = W1[:D]
    W1b = W1[D:]
    W2b = W2.astype(jnp.bfloat16)
    b1r = b1.reshape(1, D)
    b2r = b2.reshape(1, D)
    gr = gamma.reshape(1, D)
    br = beta.reshape(1, D)
    zrows = jnp.zeros((RPS, D), jnp.float32)
    ea_t = edge_attr.T

    A1 = pl.pallas_call(
        _a1_body,
        grid=(N // _NBLK,),
        in_specs=[
            pl.BlockSpec((_NBLK, D), lambda i: (i, 0)),
            pl.BlockSpec((D, D), lambda i: (0, 0)),
            pl.BlockSpec((1, D), lambda i: (0, 0)),
        ],
        out_specs=pl.BlockSpec((_NBLK, D), lambda i: (i, 0)),
        out_shape=jax.ShapeDtypeStruct((N, D), jnp.float32),
    )(H, W1a, b1r)

    aj_a = _gather_a(A1, src[0:2])
    aj_b = _gather_b(A1, src[2:5])

    ms = []
    for kc in range(CK):
        Aj, loc = (aj_a, kc) if kc < 2 else (aj_b, kc - 2)
        off = kc * (ECH // _EBLK)
        aoff = loc * (ECH // _EBLK)
        ms.append(pl.pallas_call(
            _mlp_body,
            grid=(ECH // _EBLK,),
            in_specs=[
                pl.BlockSpec((_EBLK, D), lambda i, aoff=aoff: (aoff + i, 0)),
                pl.BlockSpec((DE, _EBLK), lambda i, off=off: (0, off + i)),
                pl.BlockSpec((DE, D), lambda i: (0, 0)),
                pl.BlockSpec((D, D), lambda i: (0, 0)),
                pl.BlockSpec((1, D), lambda i: (0, 0)),
            ],
            out_specs=pl.BlockSpec((_EBLK, D), lambda i: (i, 0)),
            out_shape=jax.ShapeDtypeStruct((ECH, D), jnp.float32),
        )(Aj, ea_t, W1b, W2b, b2r))

    agg_a = _scatter_ab(ms[0], ms[1], dst[0:2], zrows)
    agg_b = _scatter_ab(ms[2], ms[3], dst[2:4], zrows)
    agg_c = _scatter_c(ms[4], dst[4:5], zrows)

    out = pl.pallas_call(
        _ln_body,
        grid=(N // _NBLK,),
        in_specs=[
            pl.BlockSpec((_NBLK, D), lambda i: (i, 0)),
            pl.BlockSpec((NC, _NBLK, D), lambda i: (0, i, 0)),
            pl.BlockSpec((NC, _NBLK, D), lambda i: (0, i, 0)),
            pl.BlockSpec((NC, _NBLK, D), lambda i: (0, i, 0)),
            pl.BlockSpec((1, D), lambda i: (0, 0)),
            pl.BlockSpec((1, D), lambda i: (0, 0)),
        ],
        out_specs=pl.BlockSpec((_NBLK, D), lambda i: (i, 0)),
        out_shape=jax.ShapeDtypeStruct((N, D), jnp.float32),
    )(H, agg_a, agg_b, agg_c, gr, br)

    return out


# speedup vs baseline: 1.3272x; 1.0041x over previous
import jax
import jax.numpy as jnp
from jax import lax
from jax.experimental import pallas as pl
from jax.experimental.pallas import tpu as pltpu
from jax.experimental.pallas import tpu_sc as plsc

N = 10000
E = 320000
D = 128
DE = 16

NC = 2
NS = 16
NW = NC * NS
CK = 5
ECH = E // CK
EPWC = ECH // NW
SBATCH = 40
SNBC = EPWC // SBATCH
NBUF = 5
NP = 10240
RPS = NP // NS

_sc_mesh = plsc.VectorSubcoreMesh(core_axis_name="c", subcore_axis_name="s")


def _make_gather(nchunks):
    def body(table_hbm, idx_hbm, out_hbm, ibuf, rows, table_s, wsems, isems):
        c = lax.axis_index("c")
        s = lax.axis_index("s")
        wid = s * NC + c
        base = wid * EPWC

        @pl.when(s < NS - 1)
        def _():
            pltpu.sync_copy(table_hbm.at[pl.ds(s * 640, 640)],
                            table_s.at[pl.ds(s * 640, 640)])

        @pl.when(s == NS - 1)
        def _():
            pltpu.sync_copy(table_hbm.at[pl.ds(9600, N - 9600)],
                            table_s.at[pl.ds(9600, N - 9600)])

        plsc.subcore_barrier()

        for kc in range(nchunks):
            def icopy(j, b, kc=kc):
                return pltpu.make_async_copy(
                    idx_hbm.at[kc, wid, j], ibuf.at[b], isems.at[b]
                )

            def wcopy(j, b, kc=kc):
                return pltpu.make_async_copy(
                    rows.at[b],
                    out_hbm.at[pl.ds(kc * ECH + base + j * SBATCH, SBATCH)],
                    wsems.at[b],
                )

            for b in range(NBUF):
                icopy(b, b).start()

            @pl.loop(0, SNBC, step=NBUF)
            def _outer(i):
                for k in range(NBUF):
                    j = i + k

                    @pl.when(j >= NBUF)
                    def _():
                        wcopy(j - NBUF, k).wait()

                    icopy(j, k).wait()
                    pltpu.sync_copy(table_s.at[ibuf.at[k]], rows.at[k])
                    nj = j + NBUF

                    @pl.when(nj < SNBC)
                    def _():
                        icopy(nj, k).start()

                    wcopy(j, k).start()

            for k in range(NBUF):
                wcopy(SNBC - NBUF + k, k).wait()

    return pl.kernel(
        body,
        out_type=jax.ShapeDtypeStruct((nchunks * ECH, D), jnp.float32),
        mesh=_sc_mesh,
        scratch_types=[
            pltpu.VMEM((NBUF, SBATCH), jnp.int32),
            pltpu.VMEM((NBUF, SBATCH, D), jnp.float32),
            pltpu.VMEM_SHARED((N, D), jnp.float32),
            pltpu.SemaphoreType.DMA((NBUF,)),
            pltpu.SemaphoreType.DMA((NBUF,)),
        ],
    )


_gather_a = _make_gather(2)
_gather_b = _make_gather(3)


def _make_scatter(nchunks, chained):
    def body(*refs):
        ms = refs[:nchunks]
        dst_hbm = refs[nchunks]
        init_hbm = refs[nchunks + 1]
        out_hbm = refs[nchunks + 2]
        ibuf, rows, acc, msems, isems = refs[nchunks + 3:]
        c = lax.axis_index("c")
        s = lax.axis_index("s")
        wid = s * NC + c
        base = wid * EPWC
        if chained:
            pltpu.sync_copy(init_hbm.at[c, pl.ds(s * RPS, RPS)],
                            acc.at[pl.ds(s * RPS, RPS)])
        else:
            pltpu.sync_copy(init_hbm, acc.at[pl.ds(s * RPS, RPS)])
        plsc.subcore_barrier()

        for kc in range(nchunks):
            def mcopy(j, b, kc=kc):
                return pltpu.make_async_copy(
                    ms[kc].at[pl.ds(base + j * SBATCH, SBATCH)],
                    rows.at[b], msems.at[b]
                )

            def icopy(j, b, kc=kc):
                return pltpu.make_async_copy(
                    dst_hbm.at[kc, wid, j], ibuf.at[b], isems.at[b]
                )

            for b in range(NBUF):
                mcopy(b, b).start()
                icopy(b, b).start()

            @pl.loop(0, SNBC, step=NBUF)
            def _outer(i):
                for k in range(NBUF):
                    j = i + k
                    mcopy(j, k).wait()
                    icopy(j, k).wait()
                    pltpu.sync_copy(rows.at[k], acc.at[ibuf.at[k]], add=True)
                    nj = j + NBUF

                    @pl.when(nj < SNBC)
                    def _():
                        mcopy(nj, k).start()
                        icopy(nj, k).start()

        plsc.subcore_barrier()
        pltpu.sync_copy(acc.at[pl.ds(s * RPS, RPS)],
                        out_hbm.at[c, pl.ds(s * RPS, RPS)])

    return pl.kernel(
        body,
        out_type=jax.ShapeDtypeStruct((NC, NP, D), jnp.float32),
        mesh=_sc_mesh,
        scratch_types=[
            pltpu.VMEM((NBUF, SBATCH), jnp.int32),
            pltpu.VMEM((NBUF, SBATCH, D), jnp.float32),
            pltpu.VMEM_SHARED((NP, D), jnp.float32),
            pltpu.SemaphoreType.DMA((NBUF,)),
            pltpu.SemaphoreType.DMA((NBUF,)),
        ],
    )


_scatter_a = _make_scatter(2, chained=False)
_scatter_b = _make_scatter(2, chained=True)
_scatter_c = _make_scatter(1, chained=True)


def _a1_body(h_ref, w_ref, b_ref, o_ref):
    o_ref[...] = (
        jnp.dot(h_ref[...], w_ref[...], preferred_element_type=jnp.float32)
        + b_ref[...]
    )


def _mlp_body(aj_ref, ea_ref, w1b_ref, w2_ref, b2_ref, o_ref):
    x = aj_ref[...] + lax.dot_general(
        ea_ref[...], w1b_ref[...], (((0,), (0,)), ((), ())),
        preferred_element_type=jnp.float32,
    )
    h = 0.5 * x * (1.0 + lax.erf(x * 0.7071067811865476))
    o_ref[...] = (
        jnp.dot(h.astype(jnp.bfloat16), w2_ref[...],
                preferred_element_type=jnp.float32)
        + b2_ref[...]
    )


def _ln_body(h_ref, agg_ref, g_ref, beta_ref, o_ref):
    x = h_ref[...] + agg_ref[0] + agg_ref[1]
    mu = jnp.mean(x, axis=-1, keepdims=True)
    xc = x - mu
    var = jnp.mean(xc * xc, axis=-1, keepdims=True)
    o_ref[...] = xc * lax.rsqrt(var + 1e-5) * g_ref[...] + beta_ref[...]


_NBLK = 1000
_EBLK = 6400


def kernel(H, edge_index, edge_attr, W1, b1, W2, b2, gamma, beta):
    src = edge_index[0].astype(jnp.int32).reshape(CK, NW, SNBC, SBATCH)
    dst = edge_index[1].astype(jnp.int32).reshape(CK, NW, SNBC, SBATCH)
    W1a = W1[:D]
    W1b = W1[D:]
    W2b = W2.astype(jnp.bfloat16)
    b1r = b1.reshape(1, D)
    b2r = b2.reshape(1, D)
    gr = gamma.reshape(1, D)
    br = beta.reshape(1, D)
    zrows = jnp.zeros((RPS, D), jnp.float32)
    ea_t = edge_attr.T

    A1 = pl.pallas_call(
        _a1_body,
        grid=(N // _NBLK,),
        in_specs=[
            pl.BlockSpec((_NBLK, D), lambda i: (i, 0)),
            pl.BlockSpec((D, D), lambda i: (0, 0)),
            pl.BlockSpec((1, D), lambda i: (0, 0)),
        ],
        out_specs=pl.BlockSpec((_NBLK, D), lambda i: (i, 0)),
        out_shape=jax.ShapeDtypeStruct((N, D), jnp.float32),
    )(H, W1a, b1r)

    aj_a = _gather_a(A1, src[0:2])
    aj_b = _gather_b(A1, src[2:5])

    ms = []
    for kc in range(CK):
        Aj, loc = (aj_a, kc) if kc < 2 else (aj_b, kc - 2)
        off = kc * (ECH // _EBLK)
        aoff = loc * (ECH // _EBLK)
        ms.append(pl.pallas_call(
            _mlp_body,
            grid=(ECH // _EBLK,),
            in_specs=[
                pl.BlockSpec((_EBLK, D), lambda i, aoff=aoff: (aoff + i, 0)),
                pl.BlockSpec((DE, _EBLK), lambda i, off=off: (0, off + i)),
                pl.BlockSpec((DE, D), lambda i: (0, 0)),
                pl.BlockSpec((D, D), lambda i: (0, 0)),
                pl.BlockSpec((1, D), lambda i: (0, 0)),
            ],
            out_specs=pl.BlockSpec((_EBLK, D), lambda i: (i, 0)),
            out_shape=jax.ShapeDtypeStruct((ECH, D), jnp.float32),
        )(Aj, ea_t, W1b, W2b, b2r))

    agg_a = _scatter_a(ms[0], ms[1], dst[0:2], zrows)
    agg_b = _scatter_b(ms[2], ms[3], dst[2:4], agg_a)
    agg = _scatter_c(ms[4], dst[4:5], agg_b)

    out = pl.pallas_call(
        _ln_body,
        grid=(N // _NBLK,),
        in_specs=[
            pl.BlockSpec((_NBLK, D), lambda i: (i, 0)),
            pl.BlockSpec((NC, _NBLK, D), lambda i: (0, i, 0)),
            pl.BlockSpec((1, D), lambda i: (0, 0)),
            pl.BlockSpec((1, D), lambda i: (0, 0)),
        ],
        out_specs=pl.BlockSpec((_NBLK, D), lambda i: (i, 0)),
        out_shape=jax.ShapeDtypeStruct((N, D), jnp.float32),
    )(H, agg, gr, br)

    return out
